# baseline (speedup 1.0000x reference)
import jax
import jax.numpy as jnp
from jax.experimental import pallas as pl
from jax.experimental.pallas import tpu as pltpu

B, T = 4096, 50
D = 256
M = 64
K = 2
TEMP = 8.0

OUT_N = 10
OUT_PAD = 128
B_BLK = 1024


def _context_coeffs():
    offsets = jnp.arange(-K, K + 1)
    pointer = jnp.arange(T, dtype=jnp.float32)
    base = jnp.clip(jnp.floor(pointer).astype(jnp.int32), 0, M - 1)
    indices = (base[:, None] + offsets[None, :]) % M
    delta = indices.astype(jnp.float32) - pointer[:, None]
    logits = -(delta ** 2) / TEMP
    w = jax.nn.softmax(logits, axis=1)
    zero = jnp.zeros((1,), jnp.float32)
    c1 = jnp.concatenate([zero, w[:T - 1, K + 1]])
    c2 = jnp.concatenate([zero, zero, w[:T - 2, K + 2]])
    return c1.reshape(1, T), c2.reshape(1, T)


def _row_sum_T(v):
    acc = v[0:8, :]
    for r in range(1, D // 8):
        acc = acc + v[8 * r:8 * (r + 1), :]
    t4 = acc[0:4, :] + acc[4:8, :]
    t2 = t4[0:2, :] + t4[2:4, :]
    return t2[0:1, :] + t2[1:2, :]


def _body(x_ref, we_ref, be_ref, wu_ref, bu_ref, g_ref, b_ref,
          wo_ref, bo_ref, s_ref, c1_ref, c2_ref, out_ref):
    s = s_ref[0, 0]
    we = we_ref[...]
    be = be_ref[...]
    wu = wu_ref[...]
    bu = bu_ref[...]
    gam = g_ref[...]
    bet = b_ref[...]
    h1 = jnp.zeros((D, B_BLK), jnp.float32)
    h2 = jnp.zeros((D, B_BLK), jnp.float32)
    for t in range(T):
        xt = x_ref[t:t + 1, :]
        inp = jnp.tanh(xt * we + be)
        context = c1_ref[0, t] * h1 + c2_ref[0, t] * h2
        combined = inp + s * context
        hm = jnp.tanh(
            jax.lax.dot_general(wu, combined + h1, (((0,), (0,)), ((), ())),
                                preferred_element_type=jnp.float32) + bu)
        mu = _row_sum_T(hm) * (1.0 / D)
        dd = hm - mu
        var = _row_sum_T(dd * dd) * (1.0 / D)
        h = dd / jnp.sqrt(var + 1e-5) * gam + bet
        h2 = h1
        h1 = h
    out_ref[...] = jax.lax.dot_general(
        wo_ref[...], h1, (((0,), (0,)), ((), ())),
        preferred_element_type=jnp.float32) + bo_ref[...]


@jax.jit
def kernel(x, W_embed, b_embed, W_update, b_update, gamma, beta,
           W_out, b_out, context_strength):
    xT = x.reshape(B, T).T
    wo_p = jnp.zeros((D, OUT_PAD), jnp.float32).at[:, :OUT_N].set(W_out)
    bo_p = jnp.zeros((OUT_PAD, 1), jnp.float32).at[:OUT_N, 0].set(b_out)
    s = jax.nn.sigmoid(context_strength).reshape(1, 1)
    c1, c2 = _context_coeffs()

    full = lambda shape: pl.BlockSpec(shape, lambda i: (0, 0))
    outT = pl.pallas_call(
        _body,
        grid=(B // B_BLK,),
        in_specs=[
            pl.BlockSpec((T, B_BLK), lambda i: (0, i)),
            full((D, 1)), full((D, 1)), full((D, D)), full((D, 1)),
            full((D, 1)), full((D, 1)), full((D, OUT_PAD)),
            full((OUT_PAD, 1)), full((1, 1)), full((1, T)), full((1, T)),
        ],
        out_specs=pl.BlockSpec((OUT_PAD, B_BLK), lambda i: (0, i)),
        out_shape=jax.ShapeDtypeStruct((OUT_PAD, B), jnp.float32),
        compiler_params=pltpu.CompilerParams(
            dimension_semantics=("arbitrary",)),
    )(xT, W_embed.reshape(D, 1), b_embed.reshape(D, 1), W_update,
      b_update.reshape(D, 1), gamma.reshape(D, 1), beta.reshape(D, 1),
      wo_p, bo_p, s, c1, c2)
    return outT.T[:, :OUT_N]


# speedup vs baseline: 1.0005x; 1.0005x over previous
import jax
import jax.numpy as jnp
from jax.experimental import pallas as pl
from jax.experimental.pallas import tpu as pltpu

B, T = 4096, 50
D = 256
M = 64
K = 2
TEMP = 8.0

OUT_N = 10
OUT_PAD = 128
B_BLK = 1024


def _context_coeffs():
    offsets = jnp.arange(-K, K + 1)
    pointer = jnp.arange(T, dtype=jnp.float32)
    base = jnp.clip(jnp.floor(pointer).astype(jnp.int32), 0, M - 1)
    indices = (base[:, None] + offsets[None, :]) % M
    delta = indices.astype(jnp.float32) - pointer[:, None]
    logits = -(delta ** 2) / TEMP
    w = jax.nn.softmax(logits, axis=1)
    zero = jnp.zeros((1,), jnp.float32)
    c1 = jnp.concatenate([zero, w[:T - 1, K + 1]])
    c2 = jnp.concatenate([zero, zero, w[:T - 2, K + 2]])
    return c1.reshape(1, T), c2.reshape(1, T)


def _row_sum_T(v):
    acc = v[0:8, :]
    for r in range(1, D // 8):
        acc = acc + v[8 * r:8 * (r + 1), :]
    t4 = acc[0:4, :] + acc[4:8, :]
    t2 = t4[0:2, :] + t4[2:4, :]
    return t2[0:1, :] + t2[1:2, :]


def _body(x_ref, we_ref, be_ref, wu_ref, bu_ref, g_ref, b_ref,
          wo_ref, bo_ref, s_ref, c1_ref, c2_ref, out_ref):
    s = s_ref[0, 0]
    we = we_ref[...]
    be = be_ref[...]
    wu = wu_ref[...]
    bu = bu_ref[...]
    gam = g_ref[...]
    bet = b_ref[...]
    h1 = jnp.zeros((D, B_BLK), jnp.float32)
    h2 = jnp.zeros((D, B_BLK), jnp.float32)
    for t in range(T):
        xt = x_ref[t:t + 1, :]
        inp = jnp.tanh(xt * we + be)
        context = c1_ref[0, t] * h1 + c2_ref[0, t] * h2
        combined = inp + s * context
        hm = jnp.tanh(
            jax.lax.dot_general(wu, combined + h1, (((0,), (0,)), ((), ())),
                                preferred_element_type=jnp.float32) + bu)
        mu = _row_sum_T(hm) * (1.0 / D)
        dd = hm - mu
        var = _row_sum_T(dd * dd) * (1.0 / D)
        h = dd / jnp.sqrt(var + 1e-5) * gam + bet
        h2 = h1
        h1 = h
    out_ref[...] = jax.lax.dot_general(
        wo_ref[...], h1, (((0,), (0,)), ((), ())),
        preferred_element_type=jnp.float32) + bo_ref[...]


@jax.jit
def kernel(x, W_embed, b_embed, W_update, b_update, gamma, beta,
           W_out, b_out, context_strength):
    xT = x.reshape(B, T).T
    wo_p = jnp.zeros((D, OUT_PAD), jnp.float32).at[:, :OUT_N].set(W_out)
    bo_p = jnp.zeros((OUT_PAD, 1), jnp.float32).at[:OUT_N, 0].set(b_out)
    s = jax.nn.sigmoid(context_strength).reshape(1, 1)
    c1, c2 = _context_coeffs()

    full = lambda shape: pl.BlockSpec(shape, lambda i: (0, 0))
    outT = pl.pallas_call(
        _body,
        grid=(B // B_BLK,),
        in_specs=[
            pl.BlockSpec((T, B_BLK), lambda i: (0, i)),
            full((D, 1)), full((D, 1)), full((D, D)), full((D, 1)),
            full((D, 1)), full((D, 1)), full((D, OUT_PAD)),
            full((OUT_PAD, 1)), full((1, 1)), full((1, T)), full((1, T)),
        ],
        out_specs=pl.BlockSpec((OUT_PAD, B_BLK), lambda i: (0, i)),
        out_shape=jax.ShapeDtypeStruct((OUT_PAD, B), jnp.float32),
        compiler_params=pltpu.CompilerParams(
            dimension_semantics=("parallel",)),
    )(xT, W_embed.reshape(D, 1), b_embed.reshape(D, 1), W_update,
      b_update.reshape(D, 1), gamma.reshape(D, 1), beta.reshape(D, 1),
      wo_p, bo_p, s, c1, c2)
    return outT.T[:, :OUT_N]


# speedup vs baseline: 1.0740x; 1.0735x over previous
import jax
import jax.numpy as jnp
from jax.experimental import pallas as pl
from jax.experimental.pallas import tpu as pltpu

B, T = 4096, 50
D = 256
M = 64
K = 2
TEMP = 8.0

OUT_N = 10
OUT_PAD = 128
B_BLK = 2048


def _context_coeffs():
    offsets = jnp.arange(-K, K + 1)
    pointer = jnp.arange(T, dtype=jnp.float32)
    base = jnp.clip(jnp.floor(pointer).astype(jnp.int32), 0, M - 1)
    indices = (base[:, None] + offsets[None, :]) % M
    delta = indices.astype(jnp.float32) - pointer[:, None]
    logits = -(delta ** 2) / TEMP
    w = jax.nn.softmax(logits, axis=1)
    zero = jnp.zeros((1,), jnp.float32)
    c1 = jnp.concatenate([zero, w[:T - 1, K + 1]])
    c2 = jnp.concatenate([zero, zero, w[:T - 2, K + 2]])
    return c1.reshape(1, T), c2.reshape(1, T)


def _row_sum_T(v):
    acc = v[0:8, :]
    for r in range(1, D // 8):
        acc = acc + v[8 * r:8 * (r + 1), :]
    t4 = acc[0:4, :] + acc[4:8, :]
    t2 = t4[0:2, :] + t4[2:4, :]
    return t2[0:1, :] + t2[1:2, :]


def _body(x_ref, we_ref, be_ref, wu_ref, bu_ref, g_ref, b_ref,
          wo_ref, bo_ref, s_ref, c1_ref, c2_ref, out_ref):
    s = s_ref[0, 0]
    we = we_ref[...]
    be = be_ref[...]
    wu = wu_ref[...]
    bu = bu_ref[...]
    gam = g_ref[...]
    bet = b_ref[...]
    h1 = jnp.zeros((D, B_BLK), jnp.float32)
    h2 = jnp.zeros((D, B_BLK), jnp.float32)
    for t in range(T):
        xt = x_ref[t:t + 1, :]
        inp = jnp.tanh(xt * we + be)
        context = c1_ref[0, t] * h1 + c2_ref[0, t] * h2
        combined = inp + s * context
        hm = jnp.tanh(
            jax.lax.dot_general(wu, combined + h1, (((0,), (0,)), ((), ())),
                                preferred_element_type=jnp.float32) + bu)
        mu = _row_sum_T(hm) * (1.0 / D)
        dd = hm - mu
        var = _row_sum_T(dd * dd) * (1.0 / D)
        h = dd / jnp.sqrt(var + 1e-5) * gam + bet
        h2 = h1
        h1 = h
    out_ref[...] = jax.lax.dot_general(
        wo_ref[...], h1, (((0,), (0,)), ((), ())),
        preferred_element_type=jnp.float32) + bo_ref[...]


@jax.jit
def kernel(x, W_embed, b_embed, W_update, b_update, gamma, beta,
           W_out, b_out, context_strength):
    xT = x.reshape(B, T).T
    wo_p = jnp.zeros((D, OUT_PAD), jnp.float32).at[:, :OUT_N].set(W_out)
    bo_p = jnp.zeros((OUT_PAD, 1), jnp.float32).at[:OUT_N, 0].set(b_out)
    s = jax.nn.sigmoid(context_strength).reshape(1, 1)
    c1, c2 = _context_coeffs()

    full = lambda shape: pl.BlockSpec(shape, lambda i: (0, 0))
    outT = pl.pallas_call(
        _body,
        grid=(B // B_BLK,),
        in_specs=[
            pl.BlockSpec((T, B_BLK), lambda i: (0, i)),
            full((D, 1)), full((D, 1)), full((D, D)), full((D, 1)),
            full((D, 1)), full((D, 1)), full((D, OUT_PAD)),
            full((OUT_PAD, 1)), full((1, 1)), full((1, T)), full((1, T)),
        ],
        out_specs=pl.BlockSpec((OUT_PAD, B_BLK), lambda i: (0, i)),
        out_shape=jax.ShapeDtypeStruct((OUT_PAD, B), jnp.float32),
        compiler_params=pltpu.CompilerParams(
            dimension_semantics=("parallel",)),
    )(xT, W_embed.reshape(D, 1), b_embed.reshape(D, 1), W_update,
      b_update.reshape(D, 1), gamma.reshape(D, 1), beta.reshape(D, 1),
      wo_p, bo_p, s, c1, c2)
    return outT.T[:, :OUT_N]


# speedup vs baseline: 1.2448x; 1.1589x over previous
import jax
import jax.numpy as jnp
from jax.experimental import pallas as pl
from jax.experimental.pallas import tpu as pltpu

B, T = 4096, 50
D = 256
M = 64
K = 2
TEMP = 8.0

OUT_N = 10
OUT_PAD = 128
B_BLK = 4096


def _context_coeffs():
    offsets = jnp.arange(-K, K + 1)
    pointer = jnp.arange(T, dtype=jnp.float32)
    base = jnp.clip(jnp.floor(pointer).astype(jnp.int32), 0, M - 1)
    indices = (base[:, None] + offsets[None, :]) % M
    delta = indices.astype(jnp.float32) - pointer[:, None]
    logits = -(delta ** 2) / TEMP
    w = jax.nn.softmax(logits, axis=1)
    zero = jnp.zeros((1,), jnp.float32)
    c1 = jnp.concatenate([zero, w[:T - 1, K + 1]])
    c2 = jnp.concatenate([zero, zero, w[:T - 2, K + 2]])
    return c1.reshape(1, T), c2.reshape(1, T)


def _row_sum_T(v):
    acc = v[0:8, :]
    for r in range(1, D // 8):
        acc = acc + v[8 * r:8 * (r + 1), :]
    t4 = acc[0:4, :] + acc[4:8, :]
    t2 = t4[0:2, :] + t4[2:4, :]
    return t2[0:1, :] + t2[1:2, :]


def _body(x_ref, we_ref, be_ref, wu_ref, bu_ref, g_ref, b_ref,
          wo_ref, bo_ref, s_ref, c1_ref, c2_ref, out_ref):
    s = s_ref[0, 0]
    we = we_ref[...]
    be = be_ref[...]
    wu = wu_ref[...]
    bu = bu_ref[...]
    gam = g_ref[...]
    bet = b_ref[...]
    h1 = jnp.zeros((D, B_BLK), jnp.float32)
    h2 = jnp.zeros((D, B_BLK), jnp.float32)
    for t in range(T):
        xt = x_ref[t:t + 1, :]
        inp = jnp.tanh(xt * we + be)
        context = c1_ref[0, t] * h1 + c2_ref[0, t] * h2
        combined = inp + s * context
        hm = jnp.tanh(
            jax.lax.dot_general(wu, combined + h1, (((0,), (0,)), ((), ())),
                                preferred_element_type=jnp.float32) + bu)
        mu = _row_sum_T(hm) * (1.0 / D)
        dd = hm - mu
        var = _row_sum_T(dd * dd) * (1.0 / D)
        h = dd / jnp.sqrt(var + 1e-5) * gam + bet
        h2 = h1
        h1 = h
    out_ref[...] = jax.lax.dot_general(
        wo_ref[...], h1, (((0,), (0,)), ((), ())),
        preferred_element_type=jnp.float32) + bo_ref[...]


@jax.jit
def kernel(x, W_embed, b_embed, W_update, b_update, gamma, beta,
           W_out, b_out, context_strength):
    xT = x.reshape(B, T).T
    wo_p = jnp.zeros((D, OUT_PAD), jnp.float32).at[:, :OUT_N].set(W_out)
    bo_p = jnp.zeros((OUT_PAD, 1), jnp.float32).at[:OUT_N, 0].set(b_out)
    s = jax.nn.sigmoid(context_strength).reshape(1, 1)
    c1, c2 = _context_coeffs()

    full = lambda shape: pl.BlockSpec(shape, lambda i: (0, 0))
    outT = pl.pallas_call(
        _body,
        grid=(B // B_BLK,),
        in_specs=[
            pl.BlockSpec((T, B_BLK), lambda i: (0, i)),
            full((D, 1)), full((D, 1)), full((D, D)), full((D, 1)),
            full((D, 1)), full((D, 1)), full((D, OUT_PAD)),
            full((OUT_PAD, 1)), full((1, 1)), full((1, T)), full((1, T)),
        ],
        out_specs=pl.BlockSpec((OUT_PAD, B_BLK), lambda i: (0, i)),
        out_shape=jax.ShapeDtypeStruct((OUT_PAD, B), jnp.float32),
        compiler_params=pltpu.CompilerParams(
            dimension_semantics=("parallel",)),
    )(xT, W_embed.reshape(D, 1), b_embed.reshape(D, 1), W_update,
      b_update.reshape(D, 1), gamma.reshape(D, 1), beta.reshape(D, 1),
      wo_p, bo_p, s, c1, c2)
    return outT.T[:, :OUT_N]
